# baseline (speedup 1.0000x reference)
import dataclasses
import functools

import jax
import jax.numpy as jnp
from jax import lax
from jax.experimental import pallas as pl
from jax.experimental.pallas import tpu as pltpu
from jax.experimental.pallas import tpu_sc as plsc

N_NODES = 10000
D = 128
N_EDGES = 320000
EPS = 1e-5

NC = 2
NS = 16
NW = NC * NS
CHUNK = 128
CPT = 81
EPT = CPT * CHUNK
EPAD = NW * EPT
NPAD = 10240
ROWS_PER_TILE = NPAD // NS

_mesh = plsc.VectorSubcoreMesh(core_axis_name="c", subcore_axis_name="s")

_sc_params = pltpu.CompilerParams()
if "needs_layout_passes" in pltpu.CompilerParams.__dataclass_fields__:
    _sc_params = dataclasses.replace(_sc_params, needs_layout_passes=False)


@jax.jit
def _sc_deg(w_flat, col2d):
    @functools.partial(
        pl.kernel,
        mesh=_mesh,
        out_type=jax.ShapeDtypeStruct((NC, NPAD, D), jnp.float32),
        compiler_params=_sc_params,
        scratch_types=[
            pltpu.VMEM((1, CHUNK), jnp.int32),
            pltpu.VMEM((EPT,), jnp.float32),
            pltpu.VMEM((CHUNK, D), jnp.float32),
            pltpu.VMEM_SHARED((NPAD, D), jnp.float32),
        ],
    )
    def k(w_hbm, col_hbm, out_hbm, cidx, wv, rbuf, acc):
        cid = lax.axis_index("c")
        sid = lax.axis_index("s")
        wid = sid * NC + cid
        pltpu.sync_copy(w_hbm.at[pl.ds(wid * EPT, EPT)], wv)

        @pl.loop(0, CHUNK)
        def _(i):
            for j in range(D // 16):
                rbuf[i, pl.ds(j * 16, 16)] = jnp.zeros((16,), jnp.float32)

        @pl.loop(0, 5)
        def _(z):
            pltpu.sync_copy(
                rbuf,
                acc.at[pl.ds(sid * ROWS_PER_TILE + z * CHUNK, CHUNK)],
            )

        plsc.subcore_barrier()

        @pl.loop(0, CPT)
        def _(ci):
            pltpu.sync_copy(col_hbm.at[pl.ds(wid * CPT + ci, 1)], cidx)

            @pl.loop(0, CHUNK)
            def _(e):
                wbc = plsc.load_gather(
                    wv, [jnp.full((16,), ci * CHUNK + e, jnp.int32)]
                )
                for j in range(D // 16):
                    rbuf[e, pl.ds(j * 16, 16)] = wbc

            pltpu.sync_copy(rbuf, acc.at[cidx.at[0]], add=True)

        plsc.subcore_barrier()
        pltpu.sync_copy(
            acc.at[pl.ds(sid * ROWS_PER_TILE, ROWS_PER_TILE)],
            out_hbm.at[cid, pl.ds(sid * ROWS_PER_TILE, ROWS_PER_TILE)],
        )

    return k(w_flat, col2d)


@jax.jit
def _sc_conv(g, row2d, col2d, w2d):
    @functools.partial(
        pl.kernel,
        mesh=_mesh,
        out_type=jax.ShapeDtypeStruct((NC, N_NODES, D), jnp.float32),
        compiler_params=_sc_params,
        scratch_types=[
            pltpu.VMEM((CHUNK, D), jnp.float32),
            pltpu.VMEM((CHUNK, D), jnp.float32),
            pltpu.VMEM((CHUNK, D), jnp.float32),
            pltpu.VMEM((1, CHUNK), jnp.int32),
            pltpu.VMEM((1, CHUNK), jnp.int32),
            pltpu.VMEM((1, CHUNK), jnp.int32),
            pltpu.VMEM((1, CHUNK), jnp.int32),
            pltpu.VMEM((1, CHUNK), jnp.int32),
            pltpu.VMEM((1, CHUNK), jnp.int32),
            pltpu.VMEM((1, CHUNK), jnp.float32),
            pltpu.VMEM((1, CHUNK), jnp.float32),
            pltpu.VMEM((1, CHUNK), jnp.float32),
            pltpu.VMEM_SHARED((N_NODES, D), jnp.float32),
            pltpu.SemaphoreType.DMA,
            pltpu.SemaphoreType.DMA,
            pltpu.SemaphoreType.DMA,
            pltpu.SemaphoreType.DMA,
            pltpu.SemaphoreType.DMA,
            pltpu.SemaphoreType.DMA,
            pltpu.SemaphoreType.DMA,
            pltpu.SemaphoreType.DMA,
            pltpu.SemaphoreType.DMA,
        ],
    )
    def k(g_hbm, row_hbm, col_hbm, w_hbm, out_hbm,
          rb0, rb1, rb2, ri0, ri1, ri2, ci0, ci1, ci2, wb0, wb1, wb2, acc,
          g0, g1, g2, s0, s1, s2, a0, a1, a2):
        cid = lax.axis_index("c")
        sid = lax.axis_index("s")
        wid = sid * NC + cid
        bufs = (rb0, rb1, rb2)
        ribs = (ri0, ri1, ri2)
        cibs = (ci0, ci1, ci2)
        wbs = (wb0, wb1, wb2)
        gsem = (g0, g1, g2)
        ssem = (s0, s1, s2)
        asem = (a0, a1, a2)

        @pl.loop(0, CHUNK)
        def _(i):
            for j in range(D // 16):
                rb0[i, pl.ds(j * 16, 16)] = jnp.zeros((16,), jnp.float32)

        @pl.when(sid < NS - 1)
        def _():
            @pl.loop(0, 5)
            def _(z):
                pltpu.sync_copy(rb0, acc.at[pl.ds(sid * 640 + z * CHUNK, CHUNK)])

        @pl.when(sid == NS - 1)
        def _():
            @pl.loop(0, 3)
            def _(z):
                pltpu.sync_copy(rb0, acc.at[pl.ds(9600 + z * CHUNK, CHUNK)])

            pltpu.sync_copy(rb0.at[pl.ds(0, 16)], acc.at[pl.ds(9984, 16)])

        plsc.subcore_barrier()

        def aux_start(cc, b):
            pltpu.make_async_copy(row_hbm.at[pl.ds(wid * CPT + cc, 1)],
                                  ribs[b], asem[b]).start()
            pltpu.make_async_copy(col_hbm.at[pl.ds(wid * CPT + cc, 1)],
                                  cibs[b], asem[b]).start()
            pltpu.make_async_copy(w_hbm.at[pl.ds(wid * CPT + cc, 1)],
                                  wbs[b], asem[b]).start()

        def aux_wait(cc, b):
            pltpu.make_async_copy(row_hbm.at[pl.ds(wid * CPT + cc, 1)],
                                  ribs[b], asem[b]).wait()
            pltpu.make_async_copy(col_hbm.at[pl.ds(wid * CPT + cc, 1)],
                                  cibs[b], asem[b]).wait()
            pltpu.make_async_copy(w_hbm.at[pl.ds(wid * CPT + cc, 1)],
                                  wbs[b], asem[b]).wait()

        def gather_start(b):
            pltpu.make_async_copy(g_hbm.at[ribs[b].at[0]], bufs[b], gsem[b]).start()

        def gather_wait(b):
            pltpu.make_async_copy(g_hbm.at[ribs[b].at[0]], bufs[b], gsem[b]).wait()

        def scatter_start(b):
            pltpu.make_async_copy(bufs[b], acc.at[cibs[b].at[0]], ssem[b]).start(add=True)

        def scatter_wait(b):
            pltpu.make_async_copy(bufs[b], acc.at[cibs[b].at[0]], ssem[b]).wait()

        def scale(b):
            buf = bufs[b]
            wbuf = wbs[b]

            @pl.loop(0, CHUNK)
            def _(e):
                wbc = plsc.load_gather(wbuf, [jnp.zeros((16,), jnp.int32),
                                              jnp.full((16,), e, jnp.int32)])
                for j in range(D // 16):
                    slc = pl.ds(j * 16, 16)
                    buf[e, slc] = buf[e, slc] * wbc

        for b in range(2):
            aux_start(b, b)
            aux_wait(b, b)
            gather_start(b)

        @pl.loop(0, CPT, step=3)
        def _(ci):
            for b in range(3):
                cc = ci + b
                nb = (b + 2) % 3
                gather_wait(b)
                scale(b)
                scatter_start(b)

                @pl.when(cc + 2 < CPT)
                def _():
                    @pl.when(cc > 0)
                    def _():
                        scatter_wait(nb)

                    aux_start(cc + 2, nb)
                    aux_wait(cc + 2, nb)
                    gather_start(nb)

        for b in range(3):
            scatter_wait(b)

        plsc.subcore_barrier()

        @pl.when(sid < NS - 1)
        def _():
            pltpu.sync_copy(acc.at[pl.ds(sid * 640, 640)],
                            out_hbm.at[cid, pl.ds(sid * 640, 640)])

        @pl.when(sid == NS - 1)
        def _():
            pltpu.sync_copy(acc.at[pl.ds(9600, 400)],
                            out_hbm.at[cid, pl.ds(9600, 400)])

    return k(g, row2d, col2d, w2d)


def _mm_body(x_ref, w_ref, o_ref):
    o_ref[...] = jnp.dot(x_ref[...], w_ref[...], preferred_element_type=jnp.float32)


@jax.jit
def _tc_mm(x, W):
    return pl.pallas_call(
        _mm_body,
        out_shape=jax.ShapeDtypeStruct((N_NODES, D), jnp.float32),
    )(x, W)


def _norm_scale_body(deg_ref, h_ref, dinv_ref, g_ref):
    deg = deg_ref[0, :N_NODES, 0:1] + deg_ref[1, :N_NODES, 0:1] + 1.0
    dinv = lax.rsqrt(deg)
    dinv_ref[...] = dinv
    g_ref[...] = h_ref[...] * dinv


@jax.jit
def _tc_norm_scale(deg_p, h):
    return pl.pallas_call(
        _norm_scale_body,
        out_shape=(
            jax.ShapeDtypeStruct((N_NODES, 1), jnp.float32),
            jax.ShapeDtypeStruct((N_NODES, D), jnp.float32),
        ),
    )(deg_p, h)


def _combine_body(s_ref, g_ref, dinv_ref, b_ref, gam_ref, bet_ref, w_ref, o_ref):
    dinv = dinv_ref[...]
    t = dinv * (s_ref[0, :N_NODES] + s_ref[1, :N_NODES] + g_ref[...]) + b_ref[...]
    m = jnp.mean(t, axis=0, keepdims=True)
    v = jnp.mean((t - m) ** 2, axis=0, keepdims=True)
    a = jnp.maximum((t - m) * lax.rsqrt(v + EPS) * gam_ref[...] + bet_ref[...], 0.0)
    o_ref[...] = dinv * jnp.dot(a, w_ref[...], preferred_element_type=jnp.float32)


@jax.jit
def _tc_combine(S_p, g, dinv, b, gam, bet, W_next):
    return pl.pallas_call(
        _combine_body,
        out_shape=jax.ShapeDtypeStruct((N_NODES, D), jnp.float32),
    )(S_p, g, dinv, b, gam, bet, W_next)


def _final_body(s_ref, g_ref, dinv_ref, b_ref, o_ref):
    o_ref[...] = dinv_ref[...] * (s_ref[0, :N_NODES] + s_ref[1, :N_NODES] + g_ref[...]) + b_ref[...]


@jax.jit
def _tc_final(S_p, g, dinv, b):
    return pl.pallas_call(
        _final_body,
        out_shape=jax.ShapeDtypeStruct((N_NODES, D), jnp.float32),
    )(S_p, g, dinv, b)


def kernel(x, edge_index, edge_attr, W1, b1, W2, b2, W3, b3, g1, be1, g2, be2):
    pad = EPAD - N_EDGES
    row_p = jnp.concatenate([edge_index[0], jnp.zeros((pad,), jnp.int32)])
    col_p = jnp.concatenate([edge_index[1], jnp.zeros((pad,), jnp.int32)])
    w_p = jnp.concatenate([edge_attr, jnp.zeros((pad,), jnp.float32)])
    row2d = row_p.reshape(EPAD // CHUNK, CHUNK)
    col2d = col_p.reshape(EPAD // CHUNK, CHUNK)
    w2d = w_p.reshape(EPAD // CHUNK, CHUNK)
    deg_p = _sc_deg(w_p, col2d)
    h1 = _tc_mm(x, W1)
    dinv, gg = _tc_norm_scale(deg_p, h1)

    S = _sc_conv(gg, row2d, col2d, w2d)
    gg = _tc_combine(S, gg, dinv, b1, g1, be1, W2)

    S = _sc_conv(gg, row2d, col2d, w2d)
    gg = _tc_combine(S, gg, dinv, b2, g2, be2, W3)

    S = _sc_conv(gg, row2d, col2d, w2d)
    return _tc_final(S, gg, dinv, b3)


# speedup vs baseline: 1.0114x; 1.0114x over previous
import dataclasses
import functools

import jax
import jax.numpy as jnp
from jax import lax
from jax.experimental import pallas as pl
from jax.experimental.pallas import tpu as pltpu
from jax.experimental.pallas import tpu_sc as plsc

N_NODES = 10000
D = 128
N_EDGES = 320000
EPS = 1e-5

NC = 2
NS = 16
NW = NC * NS
CHUNK = 128
CPT = 81
EPT = CPT * CHUNK
EPAD = NW * EPT
NPAD = 10240
ROWS_PER_TILE = NPAD // NS

_mesh = plsc.VectorSubcoreMesh(core_axis_name="c", subcore_axis_name="s")

_sc_params = pltpu.CompilerParams()
if "needs_layout_passes" in pltpu.CompilerParams.__dataclass_fields__:
    _sc_params = dataclasses.replace(_sc_params, needs_layout_passes=False)


@jax.jit
def _sc_deg(w_flat, col2d):
    @functools.partial(
        pl.kernel,
        mesh=_mesh,
        out_type=jax.ShapeDtypeStruct((NC, NPAD, D), jnp.float32),
        compiler_params=_sc_params,
        scratch_types=[
            pltpu.VMEM((1, CHUNK), jnp.int32),
            pltpu.VMEM((EPT,), jnp.float32),
            pltpu.VMEM((CHUNK, D), jnp.float32),
            pltpu.VMEM_SHARED((NPAD, D), jnp.float32),
        ],
    )
    def k(w_hbm, col_hbm, out_hbm, cidx, wv, rbuf, acc):
        cid = lax.axis_index("c")
        sid = lax.axis_index("s")
        wid = sid * NC + cid
        pltpu.sync_copy(w_hbm.at[pl.ds(wid * EPT, EPT)], wv)

        @pl.loop(0, CHUNK)
        def _(i):
            for j in range(D // 16):
                rbuf[i, pl.ds(j * 16, 16)] = jnp.zeros((16,), jnp.float32)

        @pl.loop(0, 5)
        def _(z):
            pltpu.sync_copy(
                rbuf,
                acc.at[pl.ds(sid * ROWS_PER_TILE + z * CHUNK, CHUNK)],
            )

        plsc.subcore_barrier()

        @pl.loop(0, CPT)
        def _(ci):
            pltpu.sync_copy(col_hbm.at[pl.ds(wid * CPT + ci, 1)], cidx)

            @pl.loop(0, CHUNK)
            def _(e):
                wbc = plsc.load_gather(
                    wv, [jnp.full((16,), ci * CHUNK + e, jnp.int32)]
                )
                for j in range(D // 16):
                    rbuf[e, pl.ds(j * 16, 16)] = wbc

            pltpu.sync_copy(rbuf, acc.at[cidx.at[0]], add=True)

        plsc.subcore_barrier()
        pltpu.sync_copy(
            acc.at[pl.ds(sid * ROWS_PER_TILE, ROWS_PER_TILE)],
            out_hbm.at[cid, pl.ds(sid * ROWS_PER_TILE, ROWS_PER_TILE)],
        )

    return k(w_flat, col2d)


@jax.jit
def _sc_conv(g, row2d, col2d, w2d):
    @functools.partial(
        pl.kernel,
        mesh=_mesh,
        out_type=jax.ShapeDtypeStruct((NC, N_NODES, D), jnp.float32),
        compiler_params=_sc_params,
        scratch_types=[
            pltpu.VMEM((CHUNK, D), jnp.float32),
            pltpu.VMEM((CHUNK, D), jnp.float32),
            pltpu.VMEM((CHUNK, D), jnp.float32),
            pltpu.VMEM((1, CHUNK), jnp.int32),
            pltpu.VMEM((1, CHUNK), jnp.int32),
            pltpu.VMEM((1, CHUNK), jnp.int32),
            pltpu.VMEM((1, CHUNK), jnp.int32),
            pltpu.VMEM((1, CHUNK), jnp.int32),
            pltpu.VMEM((1, CHUNK), jnp.int32),
            pltpu.VMEM((1, CHUNK), jnp.float32),
            pltpu.VMEM((1, CHUNK), jnp.float32),
            pltpu.VMEM((1, CHUNK), jnp.float32),
            pltpu.VMEM_SHARED((N_NODES, D), jnp.float32),
            pltpu.SemaphoreType.DMA,
            pltpu.SemaphoreType.DMA,
            pltpu.SemaphoreType.DMA,
            pltpu.SemaphoreType.DMA,
            pltpu.SemaphoreType.DMA,
            pltpu.SemaphoreType.DMA,
            pltpu.SemaphoreType.DMA,
            pltpu.SemaphoreType.DMA,
            pltpu.SemaphoreType.DMA,
        ],
    )
    def k(g_hbm, row_hbm, col_hbm, w_hbm, out_hbm,
          rb0, rb1, rb2, ri0, ri1, ri2, ci0, ci1, ci2, wb0, wb1, wb2, acc,
          g0, g1, g2, s0, s1, s2, a0, a1, a2):
        cid = lax.axis_index("c")
        sid = lax.axis_index("s")
        wid = sid * NC + cid
        bufs = (rb0, rb1, rb2)
        ribs = (ri0, ri1, ri2)
        cibs = (ci0, ci1, ci2)
        wbs = (wb0, wb1, wb2)
        gsem = (g0, g1, g2)
        ssem = (s0, s1, s2)
        asem = (a0, a1, a2)

        @pl.loop(0, CHUNK)
        def _(i):
            for j in range(D // 16):
                rb0[i, pl.ds(j * 16, 16)] = jnp.zeros((16,), jnp.float32)

        @pl.when(sid < NS - 1)
        def _():
            @pl.loop(0, 5)
            def _(z):
                pltpu.sync_copy(rb0, acc.at[pl.ds(sid * 640 + z * CHUNK, CHUNK)])

        @pl.when(sid == NS - 1)
        def _():
            @pl.loop(0, 3)
            def _(z):
                pltpu.sync_copy(rb0, acc.at[pl.ds(9600 + z * CHUNK, CHUNK)])

            pltpu.sync_copy(rb0.at[pl.ds(0, 16)], acc.at[pl.ds(9984, 16)])

        plsc.subcore_barrier()

        def aux_start(cc, b):
            pltpu.make_async_copy(row_hbm.at[pl.ds(wid * CPT + cc, 1)],
                                  ribs[b], asem[b]).start()
            pltpu.make_async_copy(col_hbm.at[pl.ds(wid * CPT + cc, 1)],
                                  cibs[b], asem[b]).start()
            pltpu.make_async_copy(w_hbm.at[pl.ds(wid * CPT + cc, 1)],
                                  wbs[b], asem[b]).start()

        def aux_wait(cc, b):
            pltpu.make_async_copy(row_hbm.at[pl.ds(wid * CPT + cc, 1)],
                                  ribs[b], asem[b]).wait()
            pltpu.make_async_copy(col_hbm.at[pl.ds(wid * CPT + cc, 1)],
                                  cibs[b], asem[b]).wait()
            pltpu.make_async_copy(w_hbm.at[pl.ds(wid * CPT + cc, 1)],
                                  wbs[b], asem[b]).wait()

        def gather_start(b):
            pltpu.make_async_copy(g_hbm.at[ribs[b].at[0]], bufs[b], gsem[b]).start()

        def gather_wait(b):
            pltpu.make_async_copy(g_hbm.at[ribs[b].at[0]], bufs[b], gsem[b]).wait()

        def scatter_start(b):
            pltpu.make_async_copy(bufs[b], acc.at[cibs[b].at[0]], ssem[b]).start(add=True)

        def scatter_wait(b):
            pltpu.make_async_copy(bufs[b], acc.at[cibs[b].at[0]], ssem[b]).wait()

        def scale(b):
            buf = bufs[b]
            wbuf = wbs[b]

            @pl.loop(0, CHUNK)
            def _(e):
                wbc = plsc.load_gather(wbuf, [jnp.zeros((16,), jnp.int32),
                                              jnp.full((16,), e, jnp.int32)])
                for j in range(D // 16):
                    slc = pl.ds(j * 16, 16)
                    buf[e, slc] = buf[e, slc] * wbc

        for b in range(2):
            aux_start(b, b)
            aux_wait(b, b)
            gather_start(b)

        @pl.loop(0, CPT, step=3)
        def _(ci):
            for b in range(3):
                cc = ci + b
                nb = (b + 2) % 3

                @pl.when(cc > 0)
                def _():
                    scatter_wait(nb)

                @pl.when(cc + 2 < CPT)
                def _():
                    aux_start(cc + 2, nb)

                gather_wait(b)
                scale(b)
                scatter_start(b)

                @pl.when(cc + 2 < CPT)
                def _():
                    aux_wait(cc + 2, nb)
                    gather_start(nb)

        scatter_wait((CPT - 1) % 3)

        plsc.subcore_barrier()

        @pl.when(sid < NS - 1)
        def _():
            pltpu.sync_copy(acc.at[pl.ds(sid * 640, 640)],
                            out_hbm.at[cid, pl.ds(sid * 640, 640)])

        @pl.when(sid == NS - 1)
        def _():
            pltpu.sync_copy(acc.at[pl.ds(9600, 400)],
                            out_hbm.at[cid, pl.ds(9600, 400)])

    return k(g, row2d, col2d, w2d)


def _mm_body(x_ref, w_ref, o_ref):
    o_ref[...] = jnp.dot(x_ref[...], w_ref[...], preferred_element_type=jnp.float32)


@jax.jit
def _tc_mm(x, W):
    return pl.pallas_call(
        _mm_body,
        out_shape=jax.ShapeDtypeStruct((N_NODES, D), jnp.float32),
    )(x, W)


def _norm_scale_body(deg_ref, h_ref, dinv_ref, g_ref):
    deg = deg_ref[0, :N_NODES, 0:1] + deg_ref[1, :N_NODES, 0:1] + 1.0
    dinv = lax.rsqrt(deg)
    dinv_ref[...] = dinv
    g_ref[...] = h_ref[...] * dinv


@jax.jit
def _tc_norm_scale(deg_p, h):
    return pl.pallas_call(
        _norm_scale_body,
        out_shape=(
            jax.ShapeDtypeStruct((N_NODES, 1), jnp.float32),
            jax.ShapeDtypeStruct((N_NODES, D), jnp.float32),
        ),
    )(deg_p, h)


def _combine_body(s_ref, g_ref, dinv_ref, b_ref, gam_ref, bet_ref, w_ref, o_ref):
    dinv = dinv_ref[...]
    t = dinv * (s_ref[0, :N_NODES] + s_ref[1, :N_NODES] + g_ref[...]) + b_ref[...]
    m = jnp.mean(t, axis=0, keepdims=True)
    v = jnp.mean((t - m) ** 2, axis=0, keepdims=True)
    a = jnp.maximum((t - m) * lax.rsqrt(v + EPS) * gam_ref[...] + bet_ref[...], 0.0)
    o_ref[...] = dinv * jnp.dot(a, w_ref[...], preferred_element_type=jnp.float32)


@jax.jit
def _tc_combine(S_p, g, dinv, b, gam, bet, W_next):
    return pl.pallas_call(
        _combine_body,
        out_shape=jax.ShapeDtypeStruct((N_NODES, D), jnp.float32),
    )(S_p, g, dinv, b, gam, bet, W_next)


def _final_body(s_ref, g_ref, dinv_ref, b_ref, o_ref):
    o_ref[...] = dinv_ref[...] * (s_ref[0, :N_NODES] + s_ref[1, :N_NODES] + g_ref[...]) + b_ref[...]


@jax.jit
def _tc_final(S_p, g, dinv, b):
    return pl.pallas_call(
        _final_body,
        out_shape=jax.ShapeDtypeStruct((N_NODES, D), jnp.float32),
    )(S_p, g, dinv, b)


def kernel(x, edge_index, edge_attr, W1, b1, W2, b2, W3, b3, g1, be1, g2, be2):
    pad = EPAD - N_EDGES
    row_p = jnp.concatenate([edge_index[0], jnp.zeros((pad,), jnp.int32)])
    col_p = jnp.concatenate([edge_index[1], jnp.zeros((pad,), jnp.int32)])
    w_p = jnp.concatenate([edge_attr, jnp.zeros((pad,), jnp.float32)])
    row2d = row_p.reshape(EPAD // CHUNK, CHUNK)
    col2d = col_p.reshape(EPAD // CHUNK, CHUNK)
    w2d = w_p.reshape(EPAD // CHUNK, CHUNK)
    deg_p = _sc_deg(w_p, col2d)
    h1 = _tc_mm(x, W1)
    dinv, gg = _tc_norm_scale(deg_p, h1)

    S = _sc_conv(gg, row2d, col2d, w2d)
    gg = _tc_combine(S, gg, dinv, b1, g1, be1, W2)

    S = _sc_conv(gg, row2d, col2d, w2d)
    gg = _tc_combine(S, gg, dinv, b2, g2, be2, W3)

    S = _sc_conv(gg, row2d, col2d, w2d)
    return _tc_final(S, gg, dinv, b3)


# speedup vs baseline: 1.2248x; 1.2110x over previous
import dataclasses
import functools

import jax
import jax.numpy as jnp
from jax import lax
from jax.experimental import pallas as pl
from jax.experimental.pallas import tpu as pltpu
from jax.experimental.pallas import tpu_sc as plsc

N_NODES = 10000
D = 128
N_EDGES = 320000
EPS = 1e-5

NC = 2
NS = 16
NW = NC * NS
CHUNK = 128
CPT = 80
EPT = CPT * CHUNK
EPAD = NW * EPT
NPAD = 10240
ROWS_PER_TILE = NPAD // NS

_mesh = plsc.VectorSubcoreMesh(core_axis_name="c", subcore_axis_name="s")

_sc_params = pltpu.CompilerParams()
if "needs_layout_passes" in pltpu.CompilerParams.__dataclass_fields__:
    _sc_params = dataclasses.replace(_sc_params, needs_layout_passes=False)


@jax.jit
def _sc_deg(w_flat, col2d):
    @functools.partial(
        pl.kernel,
        mesh=_mesh,
        out_type=jax.ShapeDtypeStruct((NC, NPAD, D), jnp.float32),
        compiler_params=_sc_params,
        scratch_types=[
            pltpu.VMEM((CPT, CHUNK), jnp.int32),
            pltpu.VMEM((EPT,), jnp.float32),
            pltpu.VMEM((CHUNK, D), jnp.float32),
            pltpu.VMEM_SHARED((NPAD, D), jnp.float32),
        ],
    )
    def k(w_hbm, col_hbm, out_hbm, cidx, wv, rbuf, acc):
        cid = lax.axis_index("c")
        sid = lax.axis_index("s")
        wid = sid * NC + cid
        pltpu.sync_copy(col_hbm.at[pl.ds(wid * CPT, CPT)], cidx)
        pltpu.sync_copy(w_hbm.at[pl.ds(wid * EPT, EPT)], wv)

        @pl.loop(0, CHUNK)
        def _(i):
            for j in range(D // 16):
                rbuf[i, pl.ds(j * 16, 16)] = jnp.zeros((16,), jnp.float32)

        @pl.loop(0, 5)
        def _(z):
            pltpu.sync_copy(
                rbuf,
                acc.at[pl.ds(sid * ROWS_PER_TILE + z * CHUNK, CHUNK)],
            )

        plsc.subcore_barrier()

        @pl.loop(0, CPT)
        def _(ci):
            @pl.loop(0, CHUNK)
            def _(e):
                wbc = plsc.load_gather(
                    wv, [jnp.full((16,), ci * CHUNK + e, jnp.int32)]
                )
                for j in range(D // 16):
                    rbuf[e, pl.ds(j * 16, 16)] = wbc

            pltpu.sync_copy(rbuf, acc.at[cidx.at[ci]], add=True)

        plsc.subcore_barrier()
        pltpu.sync_copy(
            acc.at[pl.ds(sid * ROWS_PER_TILE, ROWS_PER_TILE)],
            out_hbm.at[cid, pl.ds(sid * ROWS_PER_TILE, ROWS_PER_TILE)],
        )

    return k(w_flat, col2d)


@jax.jit
def _sc_conv(g, row2d, col2d, w_flat):
    @functools.partial(
        pl.kernel,
        mesh=_mesh,
        out_type=jax.ShapeDtypeStruct((NC, NPAD, D), jnp.float32),
        compiler_params=_sc_params,
        scratch_types=[
            pltpu.VMEM((CPT, CHUNK), jnp.int32),
            pltpu.VMEM((CHUNK, D), jnp.float32),
            pltpu.VMEM((CHUNK, D), jnp.float32),
            pltpu.VMEM((1, CHUNK), jnp.int32),
            pltpu.VMEM((1, CHUNK), jnp.int32),
            pltpu.VMEM((CHUNK,), jnp.float32),
            pltpu.VMEM((CHUNK,), jnp.float32),
            pltpu.VMEM_SHARED((NPAD, D), jnp.float32),
            pltpu.SemaphoreType.DMA,
            pltpu.SemaphoreType.DMA,
            pltpu.SemaphoreType.DMA,
            pltpu.SemaphoreType.DMA,
            pltpu.SemaphoreType.DMA,
            pltpu.SemaphoreType.DMA,
            pltpu.SemaphoreType.DMA,
            pltpu.SemaphoreType.DMA,
        ],
    )
    def k(g_hbm, row_hbm, col_hbm, w_hbm, out_hbm, ridx,
          rb0, rb1, cb0, cb1, wb0, wb1, acc, g0, g1, s0, s1, a0, a1, h0, h1):
        cid = lax.axis_index("c")
        sid = lax.axis_index("s")
        wid = sid * NC + cid
        bufs = (rb0, rb1)
        cbufs = (cb0, cb1)
        wbufs = (wb0, wb1)
        gsem = (g0, g1)
        hsem = (h0, h1)
        ssem = (s0, s1)
        asem = (a0, a1)
        pltpu.sync_copy(row_hbm.at[pl.ds(wid * CPT, CPT)], ridx)

        @pl.loop(0, CHUNK)
        def _(i):
            for j in range(D // 16):
                rb0[i, pl.ds(j * 16, 16)] = jnp.zeros((16,), jnp.float32)

        @pl.loop(0, 5)
        def _(z):
            pltpu.sync_copy(
                rb0,
                acc.at[pl.ds(sid * ROWS_PER_TILE + z * CHUNK, CHUNK)],
            )

        plsc.subcore_barrier()

        def fetch_start(cc, b):
            pltpu.make_async_copy(g_hbm.at[ridx.at[cc, pl.ds(0, CHUNK // 2)]],
                                  bufs[b].at[pl.ds(0, CHUNK // 2)], gsem[b]).start()
            pltpu.make_async_copy(g_hbm.at[ridx.at[cc, pl.ds(CHUNK // 2, CHUNK // 2)]],
                                  bufs[b].at[pl.ds(CHUNK // 2, CHUNK // 2)],
                                  hsem[b]).start()
            pltpu.make_async_copy(col_hbm.at[pl.ds(wid * CPT + cc, 1)],
                                  cbufs[b], asem[b]).start()
            pltpu.make_async_copy(w_hbm.at[pl.ds((wid * CPT + cc) * CHUNK, CHUNK)],
                                  wbufs[b], asem[b]).start()

        def fetch_wait(cc, b):
            pltpu.make_async_copy(g_hbm.at[ridx.at[cc, pl.ds(0, CHUNK // 2)]],
                                  bufs[b].at[pl.ds(0, CHUNK // 2)], gsem[b]).wait()
            pltpu.make_async_copy(g_hbm.at[ridx.at[cc, pl.ds(CHUNK // 2, CHUNK // 2)]],
                                  bufs[b].at[pl.ds(CHUNK // 2, CHUNK // 2)],
                                  hsem[b]).wait()
            pltpu.make_async_copy(col_hbm.at[pl.ds(wid * CPT + cc, 1)],
                                  cbufs[b], asem[b]).wait()
            pltpu.make_async_copy(w_hbm.at[pl.ds((wid * CPT + cc) * CHUNK, CHUNK)],
                                  wbufs[b], asem[b]).wait()

        def scatter_start(b):
            pltpu.make_async_copy(bufs[b], acc.at[cbufs[b].at[0]], ssem[b]).start(add=True)

        def scatter_wait(b):
            pltpu.make_async_copy(bufs[b], acc.at[cbufs[b].at[0]], ssem[b]).wait()

        def scale(b):
            buf = bufs[b]
            wbuf = wbufs[b]

            @pl.loop(0, CHUNK)
            def _(e):
                wbc = plsc.load_gather(wbuf, [jnp.full((16,), e, jnp.int32)])
                for j in range(D // 16):
                    slc = pl.ds(j * 16, 16)
                    buf[e, slc] = buf[e, slc] * wbc

        fetch_start(0, 0)

        @pl.loop(0, CPT, step=2)
        def _(ci):
            @pl.when(ci > 0)
            def _():
                scatter_wait(1)

            fetch_start(ci + 1, 1)
            fetch_wait(ci, 0)
            scale(0)
            scatter_start(0)

            scatter_wait(0)

            @pl.when(ci + 2 < CPT)
            def _():
                fetch_start(ci + 2, 0)

            fetch_wait(ci + 1, 1)
            scale(1)
            scatter_start(1)

        scatter_wait(1)

        plsc.subcore_barrier()
        pltpu.sync_copy(
            acc.at[pl.ds(sid * ROWS_PER_TILE, ROWS_PER_TILE)],
            out_hbm.at[cid, pl.ds(sid * ROWS_PER_TILE, ROWS_PER_TILE)],
        )

    return k(g, row2d, col2d, w_flat)


def _mm_body(x_ref, w_ref, o_ref):
    o_ref[...] = jnp.dot(x_ref[...], w_ref[...], preferred_element_type=jnp.float32)


@jax.jit
def _tc_mm(x, W):
    return pl.pallas_call(
        _mm_body,
        out_shape=jax.ShapeDtypeStruct((N_NODES, D), jnp.float32),
    )(x, W)


def _norm_scale_body(deg_ref, h_ref, dinv_ref, g_ref):
    deg = deg_ref[0, :N_NODES, 0:1] + deg_ref[1, :N_NODES, 0:1] + 1.0
    dinv = lax.rsqrt(deg)
    dinv_ref[...] = dinv
    g_ref[...] = h_ref[...] * dinv


@jax.jit
def _tc_norm_scale(deg_p, h):
    return pl.pallas_call(
        _norm_scale_body,
        out_shape=(
            jax.ShapeDtypeStruct((N_NODES, 1), jnp.float32),
            jax.ShapeDtypeStruct((N_NODES, D), jnp.float32),
        ),
    )(deg_p, h)


def _combine_body(s_ref, g_ref, dinv_ref, b_ref, gam_ref, bet_ref, w_ref, o_ref):
    dinv = dinv_ref[...]
    t = dinv * (s_ref[0, :N_NODES] + s_ref[1, :N_NODES] + g_ref[...]) + b_ref[...]
    m = jnp.mean(t, axis=0, keepdims=True)
    v = jnp.mean((t - m) ** 2, axis=0, keepdims=True)
    a = jnp.maximum((t - m) * lax.rsqrt(v + EPS) * gam_ref[...] + bet_ref[...], 0.0)
    o_ref[...] = dinv * jnp.dot(a, w_ref[...], preferred_element_type=jnp.float32)


@jax.jit
def _tc_combine(S_p, g, dinv, b, gam, bet, W_next):
    return pl.pallas_call(
        _combine_body,
        out_shape=jax.ShapeDtypeStruct((N_NODES, D), jnp.float32),
    )(S_p, g, dinv, b, gam, bet, W_next)


def _final_body(s_ref, g_ref, dinv_ref, b_ref, o_ref):
    o_ref[...] = dinv_ref[...] * (s_ref[0, :N_NODES] + s_ref[1, :N_NODES] + g_ref[...]) + b_ref[...]


@jax.jit
def _tc_final(S_p, g, dinv, b):
    return pl.pallas_call(
        _final_body,
        out_shape=jax.ShapeDtypeStruct((N_NODES, D), jnp.float32),
    )(S_p, g, dinv, b)


def kernel(x, edge_index, edge_attr, W1, b1, W2, b2, W3, b3, g1, be1, g2, be2):
    pad = EPAD - N_EDGES
    row_p = jnp.concatenate([edge_index[0], jnp.zeros((pad,), jnp.int32)])
    col_p = jnp.concatenate([edge_index[1], jnp.zeros((pad,), jnp.int32)])
    w_p = jnp.concatenate([edge_attr, jnp.zeros((pad,), jnp.float32)])
    row2d = row_p.reshape(EPAD // CHUNK, CHUNK)
    col2d = col_p.reshape(EPAD // CHUNK, CHUNK)
    deg_p = _sc_deg(w_p, col2d)
    h1 = _tc_mm(x, W1)
    dinv, gg = _tc_norm_scale(deg_p, h1)

    S = _sc_conv(gg, row2d, col2d, w_p)
    gg = _tc_combine(S, gg, dinv, b1, g1, be1, W2)

    S = _sc_conv(gg, row2d, col2d, w_p)
    gg = _tc_combine(S, gg, dinv, b2, g2, be2, W3)

    S = _sc_conv(gg, row2d, col2d, w_p)
    return _tc_final(S, gg, dinv, b3)
